# Initial kernel scaffold; baseline (speedup 1.0000x reference)
import functools

import jax
import jax.numpy as jnp
from jax import lax
from jax.experimental import pallas as pl
from jax.experimental.pallas import tpu as pltpu

B, S, D = 4, 2048, 1024
E, K, H = 8, 2, 4096
N = B * S
A = N * K
BLK = 256
MPAD = A + E * BLK
NBLK = MPAD // BLK
TH = 1024
RBLK = 2048


def _router_body(x_ref, wg_ref, i1_ref, i2_ref, w1_ref, w2_ref, aux_ref, acc_ref):
    i = pl.program_id(0)
    x = x_ref[...]
    wg = wg_ref[...]
    logits = lax.dot_general(x, wg, (((1,), (1,)), ((), ())),
                             preferred_element_type=jnp.float32,
                             precision=lax.Precision.HIGHEST)
    ids = lax.broadcasted_iota(jnp.int32, (RBLK, E), 1)
    m1 = jnp.max(logits, axis=1, keepdims=True)
    i1 = jnp.min(jnp.where(logits == m1, ids, E), axis=1, keepdims=True)
    neg = jnp.full_like(logits, -jnp.inf)
    l2 = jnp.where(ids == i1, neg, logits)
    m2 = jnp.max(l2, axis=1, keepdims=True)
    i2 = jnp.min(jnp.where(l2 == m2, ids, E), axis=1, keepdims=True)
    t = jnp.exp(m2 - m1)
    w1 = 1.0 / (1.0 + t)
    w2 = t / (1.0 + t)
    i1_ref[...] = i1[:, 0]
    i2_ref[...] = i2[:, 0]
    w1_ref[...] = w1[:, 0]
    w2_ref[...] = w2[:, 0]
    p = jnp.exp(logits - m1)
    p = p / jnp.sum(p, axis=1, keepdims=True)
    onehot = ((ids == i1) | (ids == i2)).astype(jnp.float32)

    @pl.when(i == 0)
    def _():
        acc_ref[...] = jnp.zeros_like(acc_ref)

    acc_ref[0, :] += jnp.sum(p, axis=0)
    acc_ref[1, :] += jnp.sum(onehot, axis=0)

    @pl.when(i == pl.num_programs(0) - 1)
    def _():
        avg_probs = acc_ref[0, :] / N
        tokens_per_expert = acc_ref[1, :] / N
        aux_ref[0, 0] = E * jnp.sum(avg_probs * tokens_per_expert)


def _router(x2d, wg):
    grid = N // RBLK
    return pl.pallas_call(
        _router_body,
        grid=(grid,),
        in_specs=[
            pl.BlockSpec((RBLK, D), lambda i: (i, 0)),
            pl.BlockSpec((E, D), lambda i: (0, 0)),
        ],
        out_specs=[
            pl.BlockSpec((RBLK,), lambda i: (i,)),
            pl.BlockSpec((RBLK,), lambda i: (i,)),
            pl.BlockSpec((RBLK,), lambda i: (i,)),
            pl.BlockSpec((RBLK,), lambda i: (i,)),
            pl.BlockSpec((1, 1), lambda i: (0, 0)),
        ],
        out_shape=[
            jax.ShapeDtypeStruct((N,), jnp.int32),
            jax.ShapeDtypeStruct((N,), jnp.int32),
            jax.ShapeDtypeStruct((N,), jnp.float32),
            jax.ShapeDtypeStruct((N,), jnp.float32),
            jax.ShapeDtypeStruct((1, 1), jnp.float32),
        ],
        scratch_shapes=[pltpu.VMEM((2, E), jnp.float32)],
    )(x2d, wg)


def _ffn_body(be_ref, x_ref, w1_ref, b1_ref, w2_ref, b2_ref, ws_ref, out_ref):
    x = x_ref[...]
    acc = jnp.zeros((BLK, D), jnp.float32)
    for t in range(H // TH):
        sl = slice(t * TH, (t + 1) * TH)
        w1t = w1_ref[0, sl, :]
        h = lax.dot_general(x, w1t, (((1,), (1,)), ((), ())),
                            preferred_element_type=jnp.float32)
        h = jnp.maximum(h + b1_ref[0, sl][None, :], 0.0).astype(jnp.bfloat16)
        w2t = w2_ref[0, :, sl]
        acc = acc + lax.dot_general(h, w2t, (((1,), (1,)), ((), ())),
                                    preferred_element_type=jnp.float32)
    acc = acc + b2_ref[0, :][None, :]
    out_ref[...] = acc * ws_ref[...].reshape(BLK, 1)


def _ffn(block_expert, xs, w1, b1, w2, b2, ws):
    return pl.pallas_call(
        _ffn_body,
        grid_spec=pltpu.PrefetchScalarGridSpec(
            num_scalar_prefetch=1,
            grid=(NBLK,),
            in_specs=[
                pl.BlockSpec((BLK, D), lambda i, be: (i, 0)),
                pl.BlockSpec((1, H, D), lambda i, be: (be[i], 0, 0)),
                pl.BlockSpec((1, H), lambda i, be: (be[i], 0)),
                pl.BlockSpec((1, D, H), lambda i, be: (be[i], 0, 0)),
                pl.BlockSpec((1, D), lambda i, be: (be[i], 0)),
                pl.BlockSpec((BLK,), lambda i, be: (i,)),
            ],
            out_specs=pl.BlockSpec((BLK, D), lambda i, be: (i, 0)),
        ),
        out_shape=jax.ShapeDtypeStruct((MPAD, D), jnp.float32),
    )(block_expert, xs, w1, b1, w2, b2, ws)


def kernel(x, Wg, W1, b1, W2, b2):
    x2d = x.reshape(N, D)
    i1, i2, w1g, w2g, aux = _router(x2d, Wg)

    e_flat = jnp.stack([i1, i2], axis=1).reshape(-1)
    oh = (e_flat[:, None] == jnp.arange(E, dtype=jnp.int32)[None, :]).astype(jnp.int32)
    cum = jnp.cumsum(oh, axis=0)
    rank = jnp.take_along_axis(cum, e_flat[:, None], axis=1)[:, 0] - 1
    counts = cum[-1]
    padded = ((counts + BLK - 1) // BLK) * BLK
    astart = jnp.concatenate([jnp.zeros(1, jnp.int32),
                              jnp.cumsum(padded)[:-1].astype(jnp.int32)])
    dst = astart[e_flat] + rank
    src_tok = jnp.zeros((MPAD,), jnp.int32).at[dst].set(
        jnp.arange(A, dtype=jnp.int32) // K)
    wsorted = jnp.zeros((MPAD,), jnp.float32).at[dst].set(
        jnp.stack([w1g, w2g], axis=1).reshape(-1))
    pos0 = dst[0::2]
    pos1 = dst[1::2]
    ends = (astart + padded).astype(jnp.int32)
    b_starts = jnp.arange(NBLK, dtype=jnp.int32) * BLK
    block_expert = jnp.minimum(
        jnp.sum((b_starts[:, None] >= ends[None, :]).astype(jnp.int32), axis=1),
        E - 1)

    xb = x2d.astype(jnp.bfloat16)
    xs = xb[src_tok]

    yw = _ffn(block_expert, xs, W1.astype(jnp.bfloat16), b1,
              W2.astype(jnp.bfloat16), b2, wsorted)

    out = yw[pos0] + yw[pos1]
    return out.reshape(B, S, D), aux.reshape(())


# speedup vs baseline: 1.3932x; 1.3932x over previous
import functools

import jax
import jax.numpy as jnp
from jax import lax
from jax.experimental import pallas as pl
from jax.experimental.pallas import tpu as pltpu
from jax.experimental.pallas import tpu_sc as plsc

B, S, D = 4, 2048, 1024
E, K, H = 8, 2, 4096
N = B * S
A = N * K
BLK = 256
MPAD = A + E * BLK
NBLK = MPAD // BLK
TH = 1024
RBLK = 2048


def _router_body(x_ref, wg_ref, i1_ref, i2_ref, w1_ref, w2_ref, aux_ref, acc_ref):
    i = pl.program_id(0)
    x = x_ref[...]
    wg = wg_ref[...]
    logits = lax.dot_general(x.astype(jnp.bfloat16), wg.astype(jnp.bfloat16),
                             (((1,), (1,)), ((), ())),
                             preferred_element_type=jnp.float32)
    cols = [logits[:, e:e + 1] for e in range(E)]
    m1 = cols[0]
    i1 = jnp.zeros((RBLK, 1), jnp.int32)
    for e in range(1, E):
        upd = cols[e] > m1
        m1 = jnp.where(upd, cols[e], m1)
        i1 = jnp.where(upd, e, i1)
    m2 = jnp.full((RBLK, 1), -jnp.inf, jnp.float32)
    i2 = jnp.full((RBLK, 1), E, jnp.int32)
    for e in range(E):
        upd = (i1 != e) & (cols[e] > m2)
        m2 = jnp.where(upd, cols[e], m2)
        i2 = jnp.where(upd, e, i2)
    t = jnp.exp(m2 - m1)
    w1 = 1.0 / (1.0 + t)
    w2 = t / (1.0 + t)
    i1_ref[...] = i1
    i2_ref[...] = i2
    w1_ref[...] = w1
    w2_ref[...] = w2
    exps = [jnp.exp(cols[e] - m1) for e in range(E)]
    psum = exps[0]
    for e in range(1, E):
        psum = psum + exps[e]

    @pl.when(i == 0)
    def _():
        for e in range(E):
            acc_ref[0, e] = 0.0
            acc_ref[1, e] = 0.0

    for e in range(E):
        acc_ref[0, e] += jnp.sum(exps[e] / psum)
        acc_ref[1, e] += jnp.sum(((i1 == e) | (i2 == e)).astype(jnp.float32))

    @pl.when(i == pl.num_programs(0) - 1)
    def _():
        aux = 0.0
        for e in range(E):
            aux += acc_ref[0, e] * acc_ref[1, e]
        aux_ref[...] = jnp.full((1, 1), E * aux / (N * N), jnp.float32)


def _router(x2d, wg):
    grid = N // RBLK
    return pl.pallas_call(
        _router_body,
        grid=(grid,),
        in_specs=[
            pl.BlockSpec((RBLK, D), lambda i: (i, 0)),
            pl.BlockSpec((E, D), lambda i: (0, 0)),
        ],
        out_specs=[
            pl.BlockSpec((RBLK, 1), lambda i: (i, 0)),
            pl.BlockSpec((RBLK, 1), lambda i: (i, 0)),
            pl.BlockSpec((RBLK, 1), lambda i: (i, 0)),
            pl.BlockSpec((RBLK, 1), lambda i: (i, 0)),
            pl.BlockSpec((1, 1), lambda i: (0, 0)),
        ],
        out_shape=[
            jax.ShapeDtypeStruct((N, 1), jnp.int32),
            jax.ShapeDtypeStruct((N, 1), jnp.int32),
            jax.ShapeDtypeStruct((N, 1), jnp.float32),
            jax.ShapeDtypeStruct((N, 1), jnp.float32),
            jax.ShapeDtypeStruct((1, 1), jnp.float32),
        ],
        scratch_shapes=[pltpu.SMEM((2, E), jnp.float32)],
    )(x2d, wg)


def _ffn_body(be_ref, x_ref, w1_ref, b1_ref, w2_ref, b2_ref, ws_ref, out_ref):
    x = x_ref[...].astype(jnp.bfloat16)
    acc = jnp.zeros((BLK, D), jnp.float32)
    for t in range(H // TH):
        sl = slice(t * TH, (t + 1) * TH)
        w1t = w1_ref[0, sl, :]
        h = lax.dot_general(x, w1t, (((1,), (1,)), ((), ())),
                            preferred_element_type=jnp.float32)
        h = jnp.maximum(h + b1_ref[0, 0, sl][None, :], 0.0).astype(jnp.bfloat16)
        w2t = w2_ref[0, :, sl]
        acc = acc + lax.dot_general(h, w2t, (((1,), (1,)), ((), ())),
                                    preferred_element_type=jnp.float32)
    acc = acc + b2_ref[0, 0, :][None, :]
    out_ref[...] = acc * ws_ref[...].reshape(BLK, 1)


def _ffn(block_expert, xs, w1, b1, w2, b2, ws):
    return pl.pallas_call(
        _ffn_body,
        grid_spec=pltpu.PrefetchScalarGridSpec(
            num_scalar_prefetch=1,
            grid=(NBLK,),
            in_specs=[
                pl.BlockSpec((BLK, D), lambda i, be: (i, 0)),
                pl.BlockSpec((1, H, D), lambda i, be: (be[i], 0, 0)),
                pl.BlockSpec((1, 1, H), lambda i, be: (be[i], 0, 0)),
                pl.BlockSpec((1, D, H), lambda i, be: (be[i], 0, 0)),
                pl.BlockSpec((1, 1, D), lambda i, be: (be[i], 0, 0)),
                pl.BlockSpec((BLK,), lambda i, be: (i,)),
            ],
            out_specs=pl.BlockSpec((BLK, D), lambda i, be: (i, 0)),
        ),
        out_shape=jax.ShapeDtypeStruct((MPAD, D), jnp.float32),
    )(block_expert, xs, w1, b1, w2, b2, ws)


NC, NS = 2, 16
NW = NC * NS
GROWS = MPAD // NW
GCH = 64
TROWS = N // NW
TCH = 32
_SC_MESH = plsc.VectorSubcoreMesh(core_axis_name="c", subcore_axis_name="s")


def _gather_body(x_hbm, idx_hbm, out_hbm, idx_v, buf, sem):
    wid = lax.axis_index("s") * NC + lax.axis_index("c")
    base = wid * GROWS
    pltpu.sync_copy(idx_hbm.at[pl.ds(base, GROWS)], idx_v)
    for c in range(GROWS // GCH):
        pltpu.async_copy(
            x_hbm.at[idx_v.at[pl.ds(c * GCH, GCH)]], buf, sem).wait()
        pltpu.sync_copy(buf, out_hbm.at[pl.ds(base + c * GCH, GCH)])


def _gather_sc(x2d, src_tok):
    return pl.kernel(
        _gather_body,
        out_type=jax.ShapeDtypeStruct((MPAD, D), jnp.float32),
        mesh=_SC_MESH,
        scratch_types=[
            pltpu.VMEM((GROWS,), jnp.int32),
            pltpu.VMEM((GCH, D), jnp.float32),
            pltpu.SemaphoreType.DMA,
        ],
    )(x2d, src_tok)


def _combine_body(y_hbm, p0_hbm, p1_hbm, out_hbm, i0_v, i1_v, b0, b1, sem0, sem1):
    wid = lax.axis_index("s") * NC + lax.axis_index("c")
    base = wid * TROWS
    pltpu.sync_copy(p0_hbm.at[pl.ds(base, TROWS)], i0_v)
    pltpu.sync_copy(p1_hbm.at[pl.ds(base, TROWS)], i1_v)
    nsl = D // 16
    for c in range(TROWS // TCH):
        cp0 = pltpu.async_copy(y_hbm.at[i0_v.at[pl.ds(c * TCH, TCH)]], b0, sem0)
        cp1 = pltpu.async_copy(y_hbm.at[i1_v.at[pl.ds(c * TCH, TCH)]], b1, sem1)
        cp0.wait()
        cp1.wait()

        def _add(j, carry):
            t = j // nsl
            s = (j % nsl) * 16
            b0[t, pl.ds(s, 16)] += b1[t, pl.ds(s, 16)]
            return carry

        lax.fori_loop(0, TCH * nsl, _add, 0)
        pltpu.sync_copy(b0, out_hbm.at[pl.ds(base + c * TCH, TCH)])


def _combine_sc(yw, pos0, pos1):
    return pl.kernel(
        _combine_body,
        out_type=jax.ShapeDtypeStruct((N, D), jnp.float32),
        mesh=_SC_MESH,
        scratch_types=[
            pltpu.VMEM((TROWS,), jnp.int32),
            pltpu.VMEM((TROWS,), jnp.int32),
            pltpu.VMEM((TCH, D), jnp.float32),
            pltpu.VMEM((TCH, D), jnp.float32),
            pltpu.SemaphoreType.DMA,
            pltpu.SemaphoreType.DMA,
        ],
    )(yw, pos0, pos1)


def kernel(x, Wg, W1, b1, W2, b2):
    x2d = x.reshape(N, D)
    i1, i2, w1g, w2g, aux = _router(x2d, Wg)

    e_flat = jnp.concatenate([i1, i2], axis=1).reshape(-1)
    oh = (e_flat[:, None] == jnp.arange(E, dtype=jnp.int32)[None, :]).astype(jnp.int32)
    cum = jnp.cumsum(oh, axis=0)
    rank = jnp.take_along_axis(cum, e_flat[:, None], axis=1)[:, 0] - 1
    counts = cum[-1]
    padded = ((counts + BLK - 1) // BLK) * BLK
    astart = jnp.concatenate([jnp.zeros(1, jnp.int32),
                              jnp.cumsum(padded)[:-1].astype(jnp.int32)])
    dst = astart[e_flat] + rank
    src_tok = jnp.zeros((MPAD,), jnp.int32).at[dst].set(
        jnp.arange(A, dtype=jnp.int32) // K)
    wsorted = jnp.zeros((MPAD,), jnp.float32).at[dst].set(
        jnp.concatenate([w1g, w2g], axis=1).reshape(-1))
    pos0 = dst[0::2]
    pos1 = dst[1::2]
    ends = (astart + padded).astype(jnp.int32)
    b_starts = jnp.arange(NBLK, dtype=jnp.int32) * BLK
    block_expert = jnp.minimum(
        jnp.sum((b_starts[:, None] >= ends[None, :]).astype(jnp.int32), axis=1),
        E - 1)

    xs = _gather_sc(x2d, src_tok)

    yw = _ffn(block_expert, xs, W1.astype(jnp.bfloat16), b1.reshape(E, 1, H),
              W2.astype(jnp.bfloat16), b2.reshape(E, 1, D), wsorted)

    out = _combine_sc(yw, pos0, pos1)
    return out.reshape(B, S, D), aux.reshape(())
